# baseline (speedup 1.0000x reference)
import functools

import jax
import jax.numpy as jnp
from jax import lax
from jax.experimental import pallas as pl
from jax.experimental.pallas import tpu as pltpu
from jax.experimental.pallas import tpu_sc as plsc


def _make_sc_add(N, EN, C):
    info = plsc.get_sparse_core_info()
    NC, NS = info.num_cores, info.num_subcores
    NW = NC * NS
    span = N // NW
    assert N % NW == 0 and span % C == 0 and EN % span == 0 and C % 16 == 0
    mesh = plsc.VectorSubcoreMesh(core_axis_name="c", subcore_axis_name="s")

    @functools.partial(
        pl.kernel,
        mesh=mesh,
        out_type=jax.ShapeDtypeStruct((N,), jnp.float32),
        scratch_types=[
            pltpu.VMEM((C,), jnp.float32),
            pltpu.VMEM((C,), jnp.float32),
        ],
    )
    def k(x_hbm, e_hbm, o_hbm, xbuf, ebuf):
        wid = lax.axis_index("s") * NC + lax.axis_index("c")
        base = wid * span
        ebase = lax.rem(base, EN)

        def step(t, carry):
            off = base + t * C
            eoff = ebase + t * C
            pltpu.sync_copy(x_hbm.at[pl.ds(off, C)], xbuf)
            pltpu.sync_copy(e_hbm.at[pl.ds(eoff, C)], ebuf)

            def add16(i, c2):
                s = pl.ds(i * 16, 16)
                xbuf[s] = xbuf[s] + ebuf[s]
                return c2

            lax.fori_loop(0, C // 16, add16, 0, unroll=8)
            pltpu.sync_copy(xbuf, o_hbm.at[pl.ds(off, C)])
            return carry

        lax.fori_loop(0, span // C, step, 0)

    return k


def kernel(x, emb):
    B, S, D = x.shape
    N = B * S * D
    EN = S * D
    k = _make_sc_add(N, EN, 16384)
    out = k(x.reshape(N), emb[:S].reshape(EN))
    return out.reshape(B, S, D)


# speedup vs baseline: 1.2816x; 1.2816x over previous
import functools

import jax
import jax.numpy as jnp
from jax import lax
from jax.experimental import pallas as pl
from jax.experimental.pallas import tpu as pltpu
from jax.experimental.pallas import tpu_sc as plsc


def _make_sc_add(N, EN, B, C):
    info = plsc.get_sparse_core_info()
    NC, NS = info.num_cores, info.num_subcores
    NW = NC * NS
    espan = EN // NW
    T = espan // C
    assert EN % NW == 0 and espan % C == 0 and C % 16 == 0 and T % 2 == 0
    mesh = plsc.VectorSubcoreMesh(core_axis_name="c", subcore_axis_name="s")

    @functools.partial(
        pl.kernel,
        mesh=mesh,
        out_type=jax.ShapeDtypeStruct((N,), jnp.float32),
        scratch_types=[
            pltpu.VMEM((C,), jnp.float32),
            pltpu.VMEM((C,), jnp.float32),
            pltpu.VMEM((C,), jnp.float32),
            pltpu.VMEM((C,), jnp.float32),
            pltpu.VMEM((C,), jnp.float32),
            pltpu.VMEM((C,), jnp.float32),
            pltpu.SemaphoreType.DMA,
            pltpu.SemaphoreType.DMA,
            pltpu.SemaphoreType.DMA,
            pltpu.SemaphoreType.DMA,
            pltpu.SemaphoreType.DMA,
            pltpu.SemaphoreType.DMA,
        ],
    )
    def k(x_hbm, e_hbm, o_hbm, xin0, xin1, xout0, xout1, eb0, eb1,
          sxin0, sxin1, so0, so1, se0, se1):
        xin = (xin0, xin1)
        xout = (xout0, xout1)
        eb = (eb0, eb1)
        sxin = (sxin0, sxin1)
        so = (so0, so1)
        se = (se0, se1)
        wid = lax.axis_index("s") * NC + lax.axis_index("c")
        webase = wid * espan

        def xoff(t, b):
            return b * EN + webase + t * C

        def eoff(t):
            return webase + t * C

        def xload(t, b, i):
            pltpu.make_async_copy(
                x_hbm.at[pl.ds(xoff(t, b), C)], xin[i], sxin[i]).start()

        def eload(t, i):
            pltpu.make_async_copy(
                e_hbm.at[pl.ds(eoff(t), C)], eb[i], se[i]).start()

        eload(0, 0)
        eload(1, 1)
        xload(0, 0, 0)
        xload(0, 1, 1)

        def body(it, carry):
            t0 = it * 2
            for tt in range(2):
                t = t0 + tt
                pltpu.make_async_copy(
                    e_hbm.at[pl.ds(eoff(t), C)], eb[tt], se[tt]).wait()
                for b in range(B):
                    i = b % 2
                    pltpu.make_async_copy(
                        x_hbm.at[pl.ds(xoff(t, b), C)], xin[i], sxin[i]).wait()
                    first_use = (tt == 0) and (b < 2)

                    def drain():
                        pltpu.make_async_copy(
                            xout[i], o_hbm.at[pl.ds(0, C)], so[i]).wait()

                    if first_use:
                        pl.when(t0 > 0)(drain)
                    else:
                        drain()

                    def add16(j, c2):
                        sl = pl.ds(j * 16, 16)
                        xout[i][sl] = xin[i][sl] + eb[tt][sl]
                        return c2

                    lax.fori_loop(0, C // 16, add16, 0, unroll=8)

                    nt = t if b < 2 else t + 1
                    nb = (b + 2) % B

                    def prefetch():
                        xload(nt, nb, i)

                    if b >= 2:
                        pl.when(nt < T)(prefetch)
                    else:
                        prefetch()

                    pltpu.make_async_copy(
                        xout[i], o_hbm.at[pl.ds(xoff(t, b), C)], so[i]).start()

                def eprefetch():
                    eload(t + 2, tt)

                pl.when(t + 2 < T)(eprefetch)
            return carry

        lax.fori_loop(0, T // 2, body, 0)

        pltpu.make_async_copy(xout[0], o_hbm.at[pl.ds(0, C)], so[0]).wait()
        pltpu.make_async_copy(xout[1], o_hbm.at[pl.ds(0, C)], so[1]).wait()

    return k


def kernel(x, emb):
    B, S, D = x.shape
    N = B * S * D
    EN = S * D
    k = _make_sc_add(N, EN, B, 16384)
    out = k(x.reshape(N), emb[:S].reshape(EN))
    return out.reshape(B, S, D)


# speedup vs baseline: 2.0619x; 1.6088x over previous
import functools

import jax
import jax.numpy as jnp
from jax import lax
from jax.experimental import pallas as pl
from jax.experimental.pallas import tpu as pltpu
from jax.experimental.pallas import tpu_sc as plsc


def _make_sc_add(N, EN, B, C):
    info = plsc.get_sparse_core_info()
    NC, NS = info.num_cores, info.num_subcores
    NW = NC * NS
    espan = EN // NW
    T = espan // C
    assert EN % NW == 0 and espan % C == 0 and C % 16 == 0 and T % 2 == 0
    mesh = plsc.VectorSubcoreMesh(core_axis_name="c", subcore_axis_name="s")

    @functools.partial(
        pl.kernel,
        mesh=mesh,
        out_type=jax.ShapeDtypeStruct((N,), jnp.float32),
        scratch_types=[
            pltpu.VMEM((C,), jnp.float32),
            pltpu.VMEM((C,), jnp.float32),
            pltpu.VMEM((C,), jnp.float32),
            pltpu.VMEM((C,), jnp.float32),
            pltpu.VMEM((C,), jnp.float32),
            pltpu.VMEM((C,), jnp.float32),
            pltpu.SemaphoreType.DMA,
            pltpu.SemaphoreType.DMA,
            pltpu.SemaphoreType.DMA,
            pltpu.SemaphoreType.DMA,
            pltpu.SemaphoreType.DMA,
            pltpu.SemaphoreType.DMA,
        ],
    )
    def k(x_hbm, e_hbm, o_hbm, xin0, xin1, xout0, xout1, eb0, eb1,
          sxin0, sxin1, so0, so1, se0, se1):
        xin = (xin0, xin1)
        xout = (xout0, xout1)
        eb = (eb0, eb1)
        sxin = (sxin0, sxin1)
        so = (so0, so1)
        se = (se0, se1)
        wid = lax.axis_index("s") * NC + lax.axis_index("c")
        webase = wid * espan

        def xoff(t, b):
            return b * EN + webase + t * C

        def eoff(t):
            return webase + t * C

        def xload(t, b, i):
            pltpu.make_async_copy(
                x_hbm.at[pl.ds(xoff(t, b), C)], xin[i], sxin[i]).start()

        def eload(t, i):
            pltpu.make_async_copy(
                e_hbm.at[pl.ds(eoff(t), C)], eb[i], se[i]).start()

        eload(0, 0)
        eload(1, 1)
        xload(0, 0, 0)
        xload(0, 1, 1)

        def body(it, carry):
            t0 = it * 2
            for tt in range(2):
                t = t0 + tt
                pltpu.make_async_copy(
                    e_hbm.at[pl.ds(eoff(t), C)], eb[tt], se[tt]).wait()
                for b in range(B):
                    i = b % 2
                    pltpu.make_async_copy(
                        x_hbm.at[pl.ds(xoff(t, b), C)], xin[i], sxin[i]).wait()
                    first_use = (tt == 0) and (b < 2)

                    def drain():
                        pltpu.make_async_copy(
                            xout[i], o_hbm.at[pl.ds(0, C)], so[i]).wait()

                    if first_use:
                        pl.when(t0 > 0)(drain)
                    else:
                        drain()

                    @plsc.parallel_loop(0, C // 16, unroll=8)
                    def add16(j):
                        sl = pl.ds(j * 16, 16)
                        xout[i][sl] = xin[i][sl] + eb[tt][sl]

                    nt = t if b < 2 else t + 1
                    nb = (b + 2) % B

                    def prefetch():
                        xload(nt, nb, i)

                    if b >= 2:
                        pl.when(nt < T)(prefetch)
                    else:
                        prefetch()

                    pltpu.make_async_copy(
                        xout[i], o_hbm.at[pl.ds(xoff(t, b), C)], so[i]).start()

                def eprefetch():
                    eload(t + 2, tt)

                pl.when(t + 2 < T)(eprefetch)
            return carry

        lax.fori_loop(0, T // 2, body, 0)

        pltpu.make_async_copy(xout[0], o_hbm.at[pl.ds(0, C)], so[0]).wait()
        pltpu.make_async_copy(xout[1], o_hbm.at[pl.ds(0, C)], so[1]).wait()

    return k


def kernel(x, emb):
    B, S, D = x.shape
    N = B * S * D
    EN = S * D
    k = _make_sc_add(N, EN, B, 16384)
    out = k(x.reshape(N), emb[:S].reshape(EN))
    return out.reshape(B, S, D)


# speedup vs baseline: 6.1335x; 2.9747x over previous
import functools

import jax
import jax.numpy as jnp
from jax import lax
from jax.experimental import pallas as pl
from jax.experimental.pallas import tpu as pltpu
from jax.experimental.pallas import tpu_sc as plsc


def _make_sc_add(S, D, B, R):
    info = plsc.get_sparse_core_info()
    NC, NS = info.num_cores, info.num_subcores
    NW = NC * NS
    rspan = S // NW
    T = rspan // R
    CD = D // 16
    assert S % NW == 0 and rspan % R == 0 and T % 2 == 0 and D % 16 == 0
    mesh = plsc.VectorSubcoreMesh(core_axis_name="c", subcore_axis_name="s")

    @functools.partial(
        pl.kernel,
        mesh=mesh,
        out_type=jax.ShapeDtypeStruct((B * S, D), jnp.float32),
        scratch_types=[
            pltpu.VMEM((R, D), jnp.float32),
            pltpu.VMEM((R, D), jnp.float32),
            pltpu.VMEM((R, D), jnp.float32),
            pltpu.VMEM((R, D), jnp.float32),
            pltpu.VMEM((R, D), jnp.float32),
            pltpu.VMEM((R, D), jnp.float32),
            pltpu.SemaphoreType.DMA,
            pltpu.SemaphoreType.DMA,
            pltpu.SemaphoreType.DMA,
            pltpu.SemaphoreType.DMA,
            pltpu.SemaphoreType.DMA,
            pltpu.SemaphoreType.DMA,
        ],
    )
    def k(x_hbm, e_hbm, o_hbm, xin0, xin1, xout0, xout1, eb0, eb1,
          sxin0, sxin1, so0, so1, se0, se1):
        xin = (xin0, xin1)
        xout = (xout0, xout1)
        eb = (eb0, eb1)
        sxin = (sxin0, sxin1)
        so = (so0, so1)
        se = (se0, se1)
        wid = lax.axis_index("s") * NC + lax.axis_index("c")
        wrbase = wid * rspan

        def xrow(t, b):
            return b * S + wrbase + t * R

        def erow(t):
            return wrbase + t * R

        def xload(t, b, i):
            pltpu.make_async_copy(
                x_hbm.at[pl.ds(xrow(t, b), R)], xin[i], sxin[i]).start()

        def eload(t, i):
            pltpu.make_async_copy(
                e_hbm.at[pl.ds(erow(t), R)], eb[i], se[i]).start()

        eload(0, 0)
        eload(1, 1)
        xload(0, 0, 0)
        xload(0, 1, 1)

        def body(it, carry):
            t0 = it * 2
            for tt in range(2):
                t = t0 + tt
                pltpu.make_async_copy(
                    e_hbm.at[pl.ds(erow(t), R)], eb[tt], se[tt]).wait()
                for b in range(B):
                    i = b % 2
                    pltpu.make_async_copy(
                        x_hbm.at[pl.ds(xrow(t, b), R)], xin[i], sxin[i]).wait()

                    def drain():
                        pltpu.make_async_copy(
                            xout[i], o_hbm.at[pl.ds(0, R)], so[i]).wait()

                    if (tt == 0) and (b < 2):
                        pl.when(t0 > 0)(drain)
                    else:
                        drain()

                    @plsc.parallel_loop(0, R * CD, unroll=8)
                    def add16(j):
                        r = j // CD
                        sl = pl.ds((j % CD) * 16, 16)
                        xout[i][r, sl] = xin[i][r, sl] + eb[tt][r, sl]

                    nt = t if b < 2 else t + 1
                    nb = (b + 2) % B

                    def prefetch():
                        xload(nt, nb, i)

                    if b >= 2:
                        pl.when(nt < T)(prefetch)
                    else:
                        prefetch()

                    pltpu.make_async_copy(
                        xout[i], o_hbm.at[pl.ds(xrow(t, b), R)], so[i]).start()

                def eprefetch():
                    eload(t + 2, tt)

                pl.when(t + 2 < T)(eprefetch)
            return carry

        lax.fori_loop(0, T // 2, body, 0)

        pltpu.make_async_copy(xout[0], o_hbm.at[pl.ds(0, R)], so[0]).wait()
        pltpu.make_async_copy(xout[1], o_hbm.at[pl.ds(0, R)], so[1]).wait()

    return k


def kernel(x, emb):
    B, S, D = x.shape
    k = _make_sc_add(S, D, B, 16)
    out = k(x.reshape(B * S, D), emb[:S])
    return out.reshape(B, S, D)
